# Initial kernel scaffold; baseline (speedup 1.0000x reference)
import jax
import jax.numpy as jnp
from jax.experimental import pallas as pl


def kernel(x, edge_index, att):
    raise NotImplementedError("write your pallas kernel here")


# speedup vs baseline: 4.1686x; 4.1686x over previous
import functools

import jax
import jax.numpy as jnp
from jax import lax
from jax.experimental import pallas as pl
from jax.experimental.pallas import tpu as pltpu
from jax.experimental.pallas import tpu_sc as plsc

N = 10000
E = 320000
D = 128
CH = 128
NCHUNK = E // CH
EB = NCHUNK
NW = 32
NPAD = 10240
STRIPE1 = NPAD // 16
ROWS_T = NPAD // 16
R0 = 63999
R1 = 64000
FRAC = 0.80078125
_SIGN = -2147483648
_MASK31 = 2147483647


def _thresh_body(att_ref, out_ref, keys_ref):
    SIGN = jnp.int32(_SIGN)
    MASK31 = jnp.int32(_MASK31)
    a = att_ref[...]
    b = lax.bitcast_convert_type(a, jnp.int32)
    keys = b ^ ((b >> 31) & MASK31)
    keys_ref[...] = keys

    def round_fn(i, uv):
        sh = 31 - i
        bit = lax.shift_left(jnp.int32(1), sh)
        low = bit - jnp.int32(1)
        t = (uv | low) ^ SIGN
        cnt = jnp.sum((keys_ref[...] <= t).astype(jnp.int32))
        return jnp.where(cnt >= jnp.int32(R1 + 1), uv, uv | bit)

    uv = lax.fori_loop(0, 32, round_fn, jnp.int32(0))
    vhi = uv ^ SIGN
    k = keys_ref[...]
    lt = k < vhi
    cnt_lt = jnp.sum(lt.astype(jnp.int32))
    maxbelow = jnp.max(jnp.where(lt, k, SIGN))
    vlo = jnp.where(cnt_lt <= jnp.int32(R0), vhi, maxbelow)

    def untr(z):
        return z ^ ((z >> 31) & MASK31)

    lo_f = lax.bitcast_convert_type(untr(vlo), jnp.float32)
    hi_f = lax.bitcast_convert_type(untr(vhi), jnp.float32)
    thr = lo_f + jnp.float32(FRAC) * (hi_f - lo_f)
    out_ref[...] = jnp.where(a > thr, a, jnp.float32(0.0))


def _worker_id():
    c = lax.axis_index("c")
    s = lax.axis_index("s")
    return c, s, s * 2 + c


def _nchunks(w):
    return (jnp.int32(NCHUNK) - w + jnp.int32(NW - 1)) // jnp.int32(NW)


def _attsum_body(masked_hbm, src_hbm, out_hbm, mv, sv, zv, acc, sem):
    c, s, w = _worker_id()
    for j in range(STRIPE1 // 16):
        zv[pl.ds(16 * j, 16)] = jnp.zeros((16,), jnp.float32)
    pltpu.sync_copy(zv, acc.at[pl.ds(STRIPE1 * s, STRIPE1)])
    plsc.subcore_barrier()

    def body(k, carry):
        base = (w + NW * k) * CH
        pltpu.sync_copy(masked_hbm.at[pl.ds(base, CH)], mv)
        pltpu.sync_copy(src_hbm.at[pl.ds(base, CH)], sv)
        pltpu.sync_copy(mv, acc.at[sv], add=True)
        return carry

    lax.fori_loop(0, _nchunks(w), body, jnp.int32(0))
    plsc.subcore_barrier()
    pltpu.sync_copy(acc.at[pl.ds(STRIPE1 * s, STRIPE1)],
                    out_hbm.at[pl.ds(c * NPAD + STRIPE1 * s, STRIPE1)])


def _attw_body(masked_hbm, src_hbm, s0_hbm, s1_hbm, out_hbm,
               mv, sv, g0, g1, wv, sem):
    c, s, w = _worker_id()

    def body(k, carry):
        base = (w + NW * k) * CH
        pltpu.sync_copy(masked_hbm.at[pl.ds(base, CH)], mv)
        pltpu.sync_copy(src_hbm.at[pl.ds(base, CH)], sv)
        pltpu.async_copy(s0_hbm.at[sv], g0, sem).wait()
        pltpu.async_copy(s1_hbm.at[sv], g1, sem).wait()
        for j in range(CH // 16):
            sl = pl.ds(16 * j, 16)
            wv[sl] = mv[sl] / (g0[sl] + g1[sl] + jnp.float32(1e-16))
        pltpu.sync_copy(wv, out_hbm.at[pl.ds(base, CH)])
        return carry

    lax.fori_loop(0, _nchunks(w), body, jnp.int32(0))


def _spmv_body(y_hbm, src_hbm, dst_hbm, w_hbm, out_hbm,
               sv, dv, wv, rows, zb, acc, sem):
    c, s, w = _worker_id()

    def zero_body(i, carry):
        for j in range(D // 16):
            zb[i, pl.ds(16 * j, 16)] = jnp.zeros((16,), jnp.float32)
        return carry

    lax.fori_loop(0, ROWS_T // 5, zero_body, jnp.int32(0))
    for j in range(5):
        pltpu.sync_copy(zb, acc.at[pl.ds(ROWS_T * s + (ROWS_T // 5) * j,
                                         ROWS_T // 5)])
    plsc.subcore_barrier()

    idx0 = jnp.zeros((16,), jnp.int32)

    def body(k, carry):
        base = (w + NW * k) * CH
        pltpu.sync_copy(src_hbm.at[pl.ds(base, CH)], sv)
        pltpu.sync_copy(dst_hbm.at[pl.ds(base, CH)], dv)
        pltpu.sync_copy(w_hbm.at[pl.ds(base, CH)], wv)
        pltpu.async_copy(y_hbm.at[sv], rows, sem).wait()

        def scale_body(i, carry2):
            wb = plsc.load_gather(wv, [idx0 + i])
            for j in range(D // 16):
                sl = pl.ds(16 * j, 16)
                rows[i, sl] = rows[i, sl] * wb
            return carry2

        lax.fori_loop(0, CH, scale_body, jnp.int32(0))
        pltpu.sync_copy(rows, acc.at[dv], add=True)
        return carry

    lax.fori_loop(0, _nchunks(w), body, jnp.int32(0))
    plsc.subcore_barrier()
    pltpu.sync_copy(acc.at[pl.ds(ROWS_T * s, ROWS_T)],
                    out_hbm.at[pl.ds(c * NPAD + ROWS_T * s, ROWS_T)])


def _combine_body(p0_ref, p1_ref, p_ref, z_ref, pn_ref, zn_ref, *, coef):
    pn = p0_ref[...] + p1_ref[...] - p_ref[...]
    pn_ref[...] = pn
    zn_ref[...] = z_ref[...] + jnp.float32(coef) * pn


_CACHE = {}


def _cached(name, builder):
    if name not in _CACHE:
        _CACHE[name] = builder()
    return _CACHE[name]


def _build_thresh():
    return pl.pallas_call(
        _thresh_body,
        out_shape=jax.ShapeDtypeStruct((EB, CH), jnp.float32),
        scratch_shapes=[pltpu.VMEM((EB, CH), jnp.int32)],
    )


def _mesh():
    return plsc.VectorSubcoreMesh(core_axis_name="c", subcore_axis_name="s")


def _build_attsum():
    return pl.kernel(
        _attsum_body,
        out_type=jax.ShapeDtypeStruct((2 * NPAD,), jnp.float32),
        mesh=_mesh(),
        scratch_types=[
            pltpu.VMEM((CH,), jnp.float32),
            pltpu.VMEM((CH,), jnp.int32),
            pltpu.VMEM((STRIPE1,), jnp.float32),
            pltpu.VMEM_SHARED((NPAD,), jnp.float32),
            pltpu.SemaphoreType.DMA,
        ],
    )


def _build_attw():
    return pl.kernel(
        _attw_body,
        out_type=jax.ShapeDtypeStruct((E,), jnp.float32),
        mesh=_mesh(),
        scratch_types=[
            pltpu.VMEM((CH,), jnp.float32),
            pltpu.VMEM((CH,), jnp.int32),
            pltpu.VMEM((CH,), jnp.float32),
            pltpu.VMEM((CH,), jnp.float32),
            pltpu.VMEM((CH,), jnp.float32),
            pltpu.SemaphoreType.DMA,
        ],
    )


def _build_spmv():
    return pl.kernel(
        _spmv_body,
        out_type=jax.ShapeDtypeStruct((2 * NPAD, D), jnp.float32),
        mesh=_mesh(),
        scratch_types=[
            pltpu.VMEM((CH,), jnp.int32),
            pltpu.VMEM((CH,), jnp.int32),
            pltpu.VMEM((CH,), jnp.float32),
            pltpu.VMEM((CH, D), jnp.float32),
            pltpu.VMEM((ROWS_T // 5, D), jnp.float32),
            pltpu.VMEM_SHARED((NPAD, D), jnp.float32),
            pltpu.SemaphoreType.DMA,
        ],
        compiler_params=pltpu.CompilerParams(needs_layout_passes=False),
    )


def _build_combine(coef):
    blk = 80
    bs = pl.BlockSpec((blk, D), lambda i: (i, 0))
    bs_p1 = pl.BlockSpec((blk, D), lambda i: (i + NPAD // blk, 0))
    return pl.pallas_call(
        functools.partial(_combine_body, coef=coef),
        grid=(N // blk,),
        in_specs=[bs, bs_p1, bs, bs],
        out_specs=[bs, bs],
        out_shape=[jax.ShapeDtypeStruct((N, D), jnp.float32)] * 2,
    )


def kernel(x, edge_index, att):
    src = edge_index[0]
    dst = edge_index[1]
    att2d = att.reshape(EB, CH)
    masked = _cached("thresh", _build_thresh)(att2d).reshape(E)
    sums = _cached("attsum", _build_attsum)(masked, src)
    attw = _cached("attw", _build_attw)(masked, src,
                                        sums[:NPAD], sums[NPAD:])
    p = x
    z = x
    for i, coef in enumerate((1.0, 0.5, 1.0 / 6.0, 1.0 / 24.0)):
        parts = _cached("spmv", _build_spmv)(p, src, dst, attw)
        p, z = _cached(f"combine{i}", lambda: _build_combine(coef))(
            parts, parts, p, z)
    return z
